# baseline (speedup 1.0000x reference)
import functools

import jax
import jax.numpy as jnp
from jax import lax
from jax.experimental import pallas as pl
from jax.experimental.pallas import tpu as pltpu
from jax.experimental.pallas import tpu_sc as plsc

D_TOKEN = 64
BATCH = 4096
HIST = 200
VOCAB = 100

NUM_CORES = 2
NUM_SUBCORES = 16
NW = NUM_CORES * NUM_SUBCORES
BCOL = BATCH // NW
N_PAIRS = HIST // 2
LANES = 16
NBG = BCOL // LANES
TROW = D_TOKEN + 1
UNROLL = 4


@functools.partial(
    pl.kernel,
    out_type=jax.ShapeDtypeStruct((HIST, D_TOKEN, BATCH), jnp.float32),
    mesh=plsc.VectorSubcoreMesh(core_axis_name="c", subcore_axis_name="s"),
    compiler_params=pltpu.CompilerParams(
        use_tc_tiling_on_sc=True, needs_layout_passes=False
    ),
    scratch_types=[
        pltpu.VMEM((VOCAB * TROW,), jnp.float32),
        pltpu.VMEM((8, BCOL), jnp.int32),
        pltpu.VMEM((2, D_TOKEN, BCOL), jnp.float32),
        pltpu.SemaphoreType.DMA((2,)),
    ],
)
def _sc_emb(table_hbm, xt_hbm, out_hbm, tab_v, idx_v, p_v, osem):
    wid = lax.axis_index("s") * NUM_CORES + lax.axis_index("c")
    col = wid * BCOL
    pltpu.sync_copy(table_hbm, tab_v)

    def compute_block(r, slot):
        for bg in range(NBG):
            iv = idx_v[r, pl.ds(bg * LANES, LANES)]
            base = iv * TROW

            @plsc.parallel_loop(0, D_TOKEN, unroll=UNROLL)
            def _(d):
                p_v[slot, d, pl.ds(bg * LANES, LANES)] = plsc.load_gather(
                    tab_v, [base + d]
                )

    def pair_body(gp, _):
        @pl.when(gp % 4 == 0)
        def _():
            pltpu.sync_copy(
                xt_hbm.at[pl.ds((gp // 4) * 8, 8), pl.ds(col, BCOL)], idx_v
            )

        for k in range(2):
            h = gp * 2 + k
            r = (gp % 4) * 2 + k

            @pl.when(gp > 0)
            def _():
                pltpu.make_async_copy(
                    p_v.at[k], out_hbm.at[h].at[:, pl.ds(col, BCOL)], osem.at[k]
                ).wait()

            compute_block(r, k)
            pltpu.async_copy(
                p_v.at[k], out_hbm.at[h].at[:, pl.ds(col, BCOL)], osem.at[k]
            )

        return ()

    lax.fori_loop(0, N_PAIRS, pair_body, ())

    for k in range(2):
        pltpu.make_async_copy(
            p_v.at[k], out_hbm.at[HIST - 2 + k].at[:, pl.ds(col, BCOL)], osem.at[k]
        ).wait()


def kernel(x, dic):
    table = jnp.concatenate([dic[-1:], dic[:-1]], axis=0)
    table = jnp.pad(table, ((0, 0), (0, TROW - D_TOKEN))).reshape(VOCAB * TROW)
    p = _sc_emb(table, x.T)
    return p.transpose(2, 0, 1)


# speedup vs baseline: 1.2005x; 1.2005x over previous
import functools

import jax
import jax.numpy as jnp
from jax import lax
from jax.experimental import pallas as pl
from jax.experimental.pallas import tpu as pltpu
from jax.experimental.pallas import tpu_sc as plsc

D_TOKEN = 64
BATCH = 4096
HIST = 200
VOCAB = 100

NUM_CORES = 2
NUM_SUBCORES = 16
NW = NUM_CORES * NUM_SUBCORES
BCOL = BATCH // NW
N_PAIRS = HIST // 2
LANES = 16
NBG = BCOL // LANES
TROW = D_TOKEN + 1
UNROLL = 16


@functools.partial(
    pl.kernel,
    out_type=jax.ShapeDtypeStruct((HIST, D_TOKEN, BATCH), jnp.float32),
    mesh=plsc.VectorSubcoreMesh(core_axis_name="c", subcore_axis_name="s"),
    compiler_params=pltpu.CompilerParams(
        use_tc_tiling_on_sc=True, needs_layout_passes=False
    ),
    scratch_types=[
        pltpu.VMEM((VOCAB * TROW,), jnp.float32),
        pltpu.VMEM((8, BCOL), jnp.int32),
        pltpu.VMEM((2, D_TOKEN, BCOL), jnp.float32),
        pltpu.SemaphoreType.DMA((2,)),
    ],
)
def _sc_emb(table_hbm, xt_hbm, out_hbm, tab_v, idx_v, p_v, osem):
    wid = lax.axis_index("s") * NUM_CORES + lax.axis_index("c")
    col = wid * BCOL
    pltpu.sync_copy(table_hbm, tab_v)

    def compute_block(r, slot):
        for bg in range(NBG):
            iv = idx_v[r, pl.ds(bg * LANES, LANES)]
            base = iv * TROW

            @plsc.parallel_loop(0, D_TOKEN, unroll=UNROLL)
            def _(d):
                p_v[slot, d, pl.ds(bg * LANES, LANES)] = plsc.load_gather(
                    tab_v, [base + d]
                )

    def pair_body(gp, _):
        @pl.when(gp % 4 == 0)
        def _():
            pltpu.sync_copy(
                xt_hbm.at[pl.ds((gp // 4) * 8, 8), pl.ds(col, BCOL)], idx_v
            )

        for k in range(2):
            h = gp * 2 + k
            r = (gp % 4) * 2 + k

            @pl.when(gp > 0)
            def _():
                pltpu.make_async_copy(
                    p_v.at[k], out_hbm.at[h].at[:, pl.ds(col, BCOL)], osem.at[k]
                ).wait()

            compute_block(r, k)
            pltpu.async_copy(
                p_v.at[k], out_hbm.at[h].at[:, pl.ds(col, BCOL)], osem.at[k]
            )

        return ()

    lax.fori_loop(0, N_PAIRS, pair_body, ())

    for k in range(2):
        pltpu.make_async_copy(
            p_v.at[k], out_hbm.at[HIST - 2 + k].at[:, pl.ds(col, BCOL)], osem.at[k]
        ).wait()


def kernel(x, dic):
    table = jnp.concatenate([dic[-1:], dic[:-1]], axis=0)
    table = jnp.pad(table, ((0, 0), (0, TROW - D_TOKEN))).reshape(VOCAB * TROW)
    p = _sc_emb(table, x.T)
    return p.transpose(2, 0, 1)


# speedup vs baseline: 1.2190x; 1.0154x over previous
import functools

import jax
import jax.numpy as jnp
from jax import lax
from jax.experimental import pallas as pl
from jax.experimental.pallas import tpu as pltpu
from jax.experimental.pallas import tpu_sc as plsc

D_TOKEN = 64
BATCH = 4096
HIST = 200
VOCAB = 100

NUM_CORES = 2
NUM_SUBCORES = 16
NW = NUM_CORES * NUM_SUBCORES
BCOL = BATCH // NW
N_PAIRS = HIST // 2
LANES = 16
NBG = BCOL // LANES
TROW = D_TOKEN + 1
UNROLL = 8


@functools.partial(
    pl.kernel,
    out_type=jax.ShapeDtypeStruct((HIST, D_TOKEN, BATCH), jnp.float32),
    mesh=plsc.VectorSubcoreMesh(core_axis_name="c", subcore_axis_name="s"),
    compiler_params=pltpu.CompilerParams(
        use_tc_tiling_on_sc=True, needs_layout_passes=False
    ),
    scratch_types=[
        pltpu.VMEM((VOCAB * TROW,), jnp.float32),
        pltpu.VMEM((8, BCOL), jnp.int32),
        pltpu.VMEM((2, D_TOKEN, BCOL), jnp.float32),
        pltpu.SemaphoreType.DMA((2,)),
    ],
)
def _sc_emb(table_hbm, xt_hbm, out_hbm, tab_v, idx_v, p_v, osem):
    wid = lax.axis_index("s") * NUM_CORES + lax.axis_index("c")
    col = wid * BCOL
    pltpu.sync_copy(table_hbm, tab_v)

    def compute_block(r, slot):
        for bg in range(NBG):
            iv = idx_v[r, pl.ds(bg * LANES, LANES)]
            base = iv * TROW

            @plsc.parallel_loop(0, D_TOKEN, unroll=UNROLL)
            def _(d):
                p_v[slot, d, pl.ds(bg * LANES, LANES)] = plsc.load_gather(
                    tab_v, [base + d]
                )

    def pair_body(gp, _):
        @pl.when(gp % 4 == 0)
        def _():
            pltpu.sync_copy(
                xt_hbm.at[pl.ds((gp // 4) * 8, 8), pl.ds(col, BCOL)], idx_v
            )

        for k in range(2):
            h = gp * 2 + k
            r = (gp % 4) * 2 + k

            @pl.when(gp > 0)
            def _():
                pltpu.make_async_copy(
                    p_v.at[k], out_hbm.at[h].at[:, pl.ds(col, BCOL)], osem.at[k]
                ).wait()

            compute_block(r, k)
            pltpu.async_copy(
                p_v.at[k], out_hbm.at[h].at[:, pl.ds(col, BCOL)], osem.at[k]
            )

        return ()

    lax.fori_loop(0, N_PAIRS, pair_body, ())

    for k in range(2):
        pltpu.make_async_copy(
            p_v.at[k], out_hbm.at[HIST - 2 + k].at[:, pl.ds(col, BCOL)], osem.at[k]
        ).wait()


def kernel(x, dic):
    table = jnp.concatenate([dic[-1:], dic[:-1]], axis=0)
    table = jnp.pad(table, ((0, 0), (0, TROW - D_TOKEN))).reshape(VOCAB * TROW)
    p = _sc_emb(table, x.T)
    return p.transpose(2, 0, 1)
